# baseline (speedup 1.0000x reference)
import functools

import jax
import jax.numpy as jnp
from jax import lax
from jax.experimental import pallas as pl
from jax.experimental.pallas import tpu as pltpu, tpu_sc as plsc

Q = 16384
K = 1000
D = 16
KPAD = 1008
NG = KPAD // 16
NW = 32
RW = Q // NW


def _sc_body(ctp_hbm, feat_hbm, out_hbm, ct_v, x_v, c2_v, out_v, sem):
    wid = lax.axis_index("s") * 2 + lax.axis_index("c")
    base = wid * RW
    pltpu.sync_copy(ctp_hbm, ct_v)
    pltpu.sync_copy(feat_hbm.at[pl.ds(base * D, RW * D)], x_v)
    lanes = lax.broadcasted_iota(jnp.int32, (16,), 0)
    lanes_f = lanes.astype(jnp.float32)

    def rbf16(v):
        c = v * 65537.0
        return c - (c - v)

    for g in range(NG):
        acc = jnp.zeros((16,), jnp.float32)
        for j in range(D):
            v = ct_v[j, pl.ds(g * 16, 16)]
            acc = acc + v * v
        c2_v[pl.ds(g * 16, 16)] = acc
    for g in range(NG):
        for j in range(D):
            sl = pl.ds(g * 16, 16)
            ct_v[j, sl] = rbf16(ct_v[j, sl])

    def row(q, res):
        xrow = rbf16(x_v[pl.ds(q * D, 16)])
        bx = [jnp.full((16,), xrow[j], jnp.float32) for j in range(D)]
        minval = jnp.full((16,), 3.0e38, jnp.float32)
        minidx = jnp.full((16,), float(KPAD), jnp.float32)
        for g in range(NG):
            acc = bx[0] * ct_v[0, pl.ds(g * 16, 16)]
            for j in range(1, D):
                acc = acc + bx[j] * ct_v[j, pl.ds(g * 16, 16)]
            score = c2_v[pl.ds(g * 16, 16)] - 2.0 * acc
            upd = score < minval
            minval = jnp.minimum(minval, score)
            minidx = jnp.where(upd, lanes_f + float(g * 16), minidx)
        def lanemin(v):
            for sh in (8, 4, 2, 1):
                p = lax.gather(
                    v, (lanes ^ sh)[:, None],
                    lax.GatherDimensionNumbers(
                        offset_dims=(), collapsed_slice_dims=(0,),
                        start_index_map=(0,)),
                    slice_sizes=(1,),
                    mode=lax.GatherScatterMode.PROMISE_IN_BOUNDS)
                v = jnp.minimum(v, p)
            return v
        m_vec = lanemin(minval)
        cand = lanemin(jnp.where(minval == m_vec, minidx, float(KPAD)))
        res = jnp.where(lanes == (q & 15), cand, res)

        @pl.when((q & 15) == 15)
        def _store():
            out_v[pl.ds(q - 15, 16)] = res.astype(jnp.int32)

        return res

    lax.fori_loop(0, RW, row, jnp.zeros((16,), jnp.float32))
    pltpu.sync_copy(out_v, out_hbm.at[pl.ds(base, RW)])


@jax.jit
def kernel(features, cluster_centers):
    pad = jnp.full((KPAD - K, D), 1e17, dtype=cluster_centers.dtype)
    ctp = jnp.concatenate([cluster_centers, pad], axis=0).T
    feat_flat = features.reshape(Q * D)

    mesh = plsc.VectorSubcoreMesh(core_axis_name="c", subcore_axis_name="s")
    k = functools.partial(
        pl.kernel,
        mesh=mesh,
        out_type=jax.ShapeDtypeStruct((Q,), jnp.int32),
        scratch_types=[
            pltpu.VMEM((D, KPAD), jnp.float32),
            pltpu.VMEM((RW * D,), jnp.float32),
            pltpu.VMEM((KPAD,), jnp.float32),
            pltpu.VMEM((RW,), jnp.int32),
            pltpu.SemaphoreType.DMA,
        ],
    )(_sc_body)
    return k(ctp, feat_flat)


# speedup vs baseline: 59.9793x; 59.9793x over previous
import jax
import jax.numpy as jnp
from jax.experimental import pallas as pl
from jax.experimental.pallas import tpu as pltpu

Q = 16384
K = 1000
D = 16
KP = 1024
BQ = 1024
GRID = Q // BQ
NCH = KP // 8


def _body(c_ref, xt_ref, out_ref, cm2_ref, c2_ref, x2_ref, mm_ref):
    i = pl.program_id(0)

    @pl.when(i == 0)
    def _prep():
        c = c_ref[...]
        c2_ref[...] = jnp.sum(c * c, axis=1, keepdims=True)
        cm2_ref[...] = -2.0 * c

    xt = xt_ref[...]
    x2_ref[i % 2] = jnp.sum(xt * xt, axis=0, keepdims=True)
    mm_ref[i % 2] = jnp.dot(cm2_ref[...], xt,
                            preferred_element_type=jnp.float32)

    j = (i - 1) % 2
    x2b = jnp.broadcast_to(x2_ref[j], (8, BQ))
    big = jnp.full((8, BQ), 3.0e38, jnp.float32)
    cm = [big, big]
    cc = [jnp.zeros((8, BQ), jnp.float32)] * 2
    for r in range(NCH):
        p = r & 1
        t = x2b + c2_ref[pl.ds(8 * r, 8), :]
        d2 = t + mm_ref[j, pl.ds(8 * r, 8), :]
        upd = d2 < cm[p]
        cm[p] = jnp.minimum(cm[p], d2)
        cc[p] = jnp.where(upd, float(r), cc[p])
    curmin = jnp.minimum(cm[0], cm[1])
    m = jnp.min(curmin, axis=0, keepdims=True)
    mc = jnp.maximum(m, 0.0)
    srow = jax.lax.broadcasted_iota(jnp.int32, (8, BQ), 0).astype(jnp.float32)
    big_idx = jnp.full((8, BQ), float(KP), jnp.float32)
    cand0 = jnp.where(cm[0] <= mc, cc[0] * 8.0 + srow, big_idx)
    cand1 = jnp.where(cm[1] <= mc, cc[1] * 8.0 + srow, big_idx)
    cand = jnp.minimum(cand0, cand1)
    out_ref[0, 0, :] = jnp.min(cand, axis=0).astype(jnp.int32)


@jax.jit
def kernel(features, cluster_centers):
    pad = jnp.full((KP - K, D), 1e17, dtype=cluster_centers.dtype)
    c = jnp.concatenate([cluster_centers, pad], axis=0)
    xt = features.T

    out = pl.pallas_call(
        _body,
        grid=(GRID + 1,),
        in_specs=[
            pl.BlockSpec((KP, D), lambda i: (0, 0)),
            pl.BlockSpec((D, BQ), lambda i: (0, jnp.minimum(i, GRID - 1))),
        ],
        out_specs=pl.BlockSpec((1, 1, BQ), lambda i: (jnp.maximum(i - 1, 0), 0, 0)),
        out_shape=jax.ShapeDtypeStruct((GRID, 1, BQ), jnp.int32),
        scratch_shapes=[
            pltpu.VMEM((KP, D), jnp.float32),
            pltpu.VMEM((KP, 1), jnp.float32),
            pltpu.VMEM((2, 1, BQ), jnp.float32),
            pltpu.VMEM((2, KP, BQ), jnp.float32),
        ],
    )(c, xt)
    return out.reshape(Q)
